# Initial kernel scaffold; baseline (speedup 1.0000x reference)
import jax
import jax.numpy as jnp
from jax.experimental import pallas as pl


def kernel(x, tables):
    raise NotImplementedError("write your pallas kernel here")


# speedup vs baseline: 3.2110x; 3.2110x over previous
import functools

import jax
import jax.numpy as jnp
from jax import lax
from jax.experimental import pallas as pl
from jax.experimental.pallas import tpu as pltpu
from jax.experimental.pallas import tpu_sc as plsc

CHUNK = 128


@functools.partial(jax.jit, static_argnums=(2,))
def _sc_gather(table, idx, rows):
    emb = table.shape[1]
    info = plsc.get_sparse_core_info()
    nw = info.num_cores * info.num_subcores
    rows_per_w = rows // nw
    n_chunks = rows_per_w // CHUNK
    assert rows_per_w * nw == rows and n_chunks * CHUNK == rows_per_w

    mesh = plsc.VectorSubcoreMesh(core_axis_name="c", subcore_axis_name="s")

    @functools.partial(
        pl.kernel,
        mesh=mesh,
        out_type=jax.ShapeDtypeStruct((rows, emb), jnp.float32),
        scratch_types=[
            pltpu.VMEM((rows_per_w,), jnp.int32),
            pltpu.VMEM((CHUNK, emb), jnp.float32),
            pltpu.SemaphoreType.DMA,
        ],
    )
    def k(table_hbm, idx_hbm, out_hbm, idx_v, rows_v, sem):
        wid = lax.axis_index("s") * info.num_cores + lax.axis_index("c")
        base = wid * rows_per_w
        pltpu.sync_copy(idx_hbm.at[pl.ds(base, rows_per_w)], idx_v)

        def step(j, carry):
            off = j * CHUNK
            pltpu.async_copy(
                table_hbm.at[idx_v.at[pl.ds(off, CHUNK)]], rows_v, sem
            ).wait()
            pltpu.sync_copy(rows_v, out_hbm.at[pl.ds(base + off, CHUNK)])
            return carry

        lax.fori_loop(0, n_chunks, step, 0)

    return k(table, idx)


def kernel(x, tables):
    b, l, nf = x.shape
    vocab, emb = tables[0].shape
    table = jnp.concatenate(tables, axis=0)
    offs = jnp.arange(nf, dtype=jnp.int32) * vocab
    idx = (x.astype(jnp.int32) + offs).reshape(-1)
    out = _sc_gather(table, idx, b * l * nf)
    return out.reshape(b, l, nf * emb)


# speedup vs baseline: 3.6371x; 1.1327x over previous
import functools

import jax
import jax.numpy as jnp
from jax import lax
from jax.experimental import pallas as pl
from jax.experimental.pallas import tpu as pltpu
from jax.experimental.pallas import tpu_sc as plsc

CHUNK = 128
NBUF = 4


@functools.partial(jax.jit, static_argnums=(2,))
def _sc_gather(table, idx, rows):
    emb = table.shape[1]
    info = plsc.get_sparse_core_info()
    nw = info.num_cores * info.num_subcores
    rows_per_w = rows // nw
    n_chunks = rows_per_w // CHUNK
    n_groups = n_chunks // NBUF
    assert rows_per_w * nw == rows and n_chunks * CHUNK == rows_per_w

    mesh = plsc.VectorSubcoreMesh(core_axis_name="c", subcore_axis_name="s")

    @functools.partial(
        pl.kernel,
        mesh=mesh,
        out_type=jax.ShapeDtypeStruct((rows, emb), jnp.float32),
        scratch_types=[
            pltpu.VMEM((rows_per_w,), jnp.int32),
            *[pltpu.VMEM((CHUNK, emb), jnp.float32) for _ in range(NBUF)],
            *[pltpu.SemaphoreType.DMA for _ in range(2 * NBUF)],
        ],
    )
    def k(table_hbm, idx_hbm, out_hbm, idx_v, *bufs_sems):
        bufs = bufs_sems[:NBUF]
        gsems = bufs_sems[NBUF : 2 * NBUF]
        osems = bufs_sems[2 * NBUF :]
        wid = lax.axis_index("s") * info.num_cores + lax.axis_index("c")
        base = wid * rows_per_w
        pltpu.sync_copy(idx_hbm.at[pl.ds(base, rows_per_w)], idx_v)

        def gather(j, b):
            off = j * CHUNK
            return pltpu.async_copy(
                table_hbm.at[idx_v.at[pl.ds(off, CHUNK)]], bufs[b], gsems[b]
            )

        def write(j, b):
            off = j * CHUNK
            return pltpu.async_copy(
                bufs[b], out_hbm.at[pl.ds(base + off, CHUNK)], osems[b]
            )

        def group(g, carry):
            j0 = g * NBUF
            gs = [gather(j0 + b, b) for b in range(NBUF)]
            ws = []
            for b in range(NBUF):
                gs[b].wait()
                ws.append(write(j0 + b, b))
            for b in range(NBUF):
                ws[b].wait()
            return carry

        lax.fori_loop(0, n_groups, group, 0)
        for j in range(n_groups * NBUF, n_chunks):
            gather(j, 0).wait()
            write(j, 0).wait()

    return k(table, idx)


def kernel(x, tables):
    b, l, nf = x.shape
    vocab, emb = tables[0].shape
    table = jnp.concatenate(tables, axis=0)
    offs = jnp.arange(nf, dtype=jnp.int32) * vocab
    idx = (x.astype(jnp.int32) + offs).reshape(-1)
    out = _sc_gather(table, idx, b * l * nf)
    return out.reshape(b, l, nf * emb)


# speedup vs baseline: 3.6870x; 1.0137x over previous
import functools

import jax
import jax.numpy as jnp
from jax import lax
from jax.experimental import pallas as pl
from jax.experimental.pallas import tpu as pltpu
from jax.experimental.pallas import tpu_sc as plsc

CHUNK = 128
NBUF = 4


@functools.partial(jax.jit, static_argnums=(2,))
def _sc_gather(table, idx, rows):
    emb = table.shape[1]
    info = plsc.get_sparse_core_info()
    nw = info.num_cores * info.num_subcores
    rows_per_w = rows // nw
    n_chunks = rows_per_w // CHUNK
    n_groups = n_chunks // NBUF
    assert rows_per_w * nw == rows and n_chunks * CHUNK == rows_per_w

    mesh = plsc.VectorSubcoreMesh(core_axis_name="c", subcore_axis_name="s")

    @functools.partial(
        pl.kernel,
        mesh=mesh,
        out_type=jax.ShapeDtypeStruct((rows, emb), jnp.float32),
        scratch_types=[
            pltpu.VMEM((rows_per_w,), jnp.int32),
            *[pltpu.VMEM((CHUNK, emb), jnp.float32) for _ in range(NBUF)],
            *[pltpu.SemaphoreType.DMA for _ in range(2 * NBUF)],
        ],
    )
    def k(table_hbm, idx_hbm, out_hbm, idx_v, *bufs_sems):
        bufs = bufs_sems[:NBUF]
        gsems = bufs_sems[NBUF : 2 * NBUF]
        osems = bufs_sems[2 * NBUF :]
        wid = lax.axis_index("s") * info.num_cores + lax.axis_index("c")
        base = wid * rows_per_w
        pltpu.sync_copy(idx_hbm.at[pl.ds(base, rows_per_w)], idx_v)

        def gather(j, b):
            off = j * CHUNK
            pltpu.async_copy(
                table_hbm.at[idx_v.at[pl.ds(off, CHUNK)]], bufs[b], gsems[b]
            )

        def gather_wait(j, b):
            off = j * CHUNK
            pltpu.make_async_copy(
                table_hbm.at[idx_v.at[pl.ds(off, CHUNK)]], bufs[b], gsems[b]
            ).wait()

        def write(j, b):
            off = j * CHUNK
            pltpu.async_copy(
                bufs[b], out_hbm.at[pl.ds(base + off, CHUNK)], osems[b]
            )

        def write_wait(j, b):
            off = j * CHUNK
            pltpu.make_async_copy(
                bufs[b], out_hbm.at[pl.ds(base + off, CHUNK)], osems[b]
            ).wait()

        for b in range(NBUF):
            gather(b, b)

        def group(g, carry):
            j0 = g * NBUF
            for b in range(NBUF):
                gather_wait(j0 + b, b)
                write(j0 + b, b)
            for b in range(NBUF):
                write_wait(j0 + b, b)
                gather(j0 + NBUF + b, b)
            return carry

        lax.fori_loop(0, n_groups - 1, group, 0)
        j0 = (n_groups - 1) * NBUF
        for b in range(NBUF):
            gather_wait(j0 + b, b)
            write(j0 + b, b)
        for b in range(NBUF):
            write_wait(j0 + b, b)
        for j in range(n_groups * NBUF, n_chunks):
            gather(j, 0)
            gather_wait(j, 0)
            write(j, 0)
            write_wait(j, 0)

    return k(table, idx)


def kernel(x, tables):
    b, l, nf = x.shape
    vocab, emb = tables[0].shape
    table = jnp.concatenate(tables, axis=0)
    offs = jnp.arange(nf, dtype=jnp.int32) * vocab
    idx = (x.astype(jnp.int32) + offs).reshape(-1)
    out = _sc_gather(table, idx, b * l * nf)
    return out.reshape(b, l, nf * emb)
